# baseline (speedup 1.0000x reference)
import functools

import jax
import jax.numpy as jnp
from jax import lax
from jax.experimental import pallas as pl
from jax.experimental.pallas import tpu as pltpu
from jax.experimental.pallas import tpu_sc as plsc

N_NODES = 10000
N_EDGES = 320000
D_FEAT = 128
BATCH = 2
LANES = 16

NUM_TILES = 16
CHUNK = 128
N_STAGES = 8
SCHUNKS = 20
E_PER_TILE = N_STAGES * SCHUNKS * CHUNK
E_RAW_PER_TILE = N_EDGES // NUM_TILES
PAD_PER_TILE = E_PER_TILE - E_RAW_PER_TILE

TRASH = 8
ACC_ROWS = N_NODES + TRASH
ROWS_PER_TILE = 632
LAST_ROWS = ACC_ROWS - 15 * ROWS_PER_TILE
LAST_VALID = N_NODES - 15 * ROWS_PER_TILE


def _sc_body(t_hbm, row_hbm, col_hbm, out_hbm,
             row_v, col_v, row_v2, col_v2, gbuf, gbuf2, acc,
             gsem, gsem2, isem, isem2):
    c = lax.axis_index("c")
    s = lax.axis_index("s")
    nbase = s * ROWS_PER_TILE

    def zrow(i, carry):
        for j in range(D_FEAT // LANES):
            gbuf[i, pl.ds(j * LANES, LANES)] = jnp.zeros((LANES,), jnp.float32)
        return carry
    lax.fori_loop(0, CHUNK, zrow, 0)

    @pl.when(s < NUM_TILES - 1)
    def _():
        for i in range(4):
            pltpu.sync_copy(gbuf, acc.at[pl.ds(nbase + i * CHUNK, CHUNK)])
        pltpu.sync_copy(gbuf.at[pl.ds(0, ROWS_PER_TILE - 4 * CHUNK)],
                        acc.at[pl.ds(nbase + 4 * CHUNK, ROWS_PER_TILE - 4 * CHUNK)])

    @pl.when(s == NUM_TILES - 1)
    def _():
        for i in range(4):
            pltpu.sync_copy(gbuf, acc.at[pl.ds(nbase + i * CHUNK, CHUNK)])
        pltpu.sync_copy(gbuf.at[pl.ds(0, LAST_ROWS - 4 * CHUNK)],
                        acc.at[pl.ds(nbase + 4 * CHUNK, LAST_ROWS - 4 * CHUNK)])

    plsc.subcore_barrier()

    def gather(rv, k, buf, sem):
        pltpu.async_copy(t_hbm.at[rv.at[k]], buf, sem)

    def gwait(rv, k, buf, sem):
        pltpu.make_async_copy(t_hbm.at[rv.at[k]], buf, sem).wait()

    def iload(q, rv, cv, sem):
        pltpu.async_copy(row_hbm.at[c, s, q], rv, sem)
        pltpu.async_copy(col_hbm.at[s, q], cv, sem)

    def iwait(q, rv, cv, sem):
        pltpu.make_async_copy(row_hbm.at[c, s, q], rv, sem).wait()
        pltpu.make_async_copy(col_hbm.at[s, q], cv, sem).wait()

    bufs = [(row_v, col_v, isem), (row_v2, col_v2, isem2)]
    iload(0, *bufs[0])
    for q in range(N_STAGES):
        rv, cv, isem_q = bufs[q % 2]
        iwait(q, rv, cv, isem_q)
        if q + 1 < N_STAGES:
            iload(q + 1, *bufs[(q + 1) % 2])

        gather(rv, 0, gbuf, gsem)

        def pair_body(g, carry):
            k0 = 2 * g
            gather(rv, k0 + 1, gbuf2, gsem2)
            gwait(rv, k0, gbuf, gsem)
            pltpu.sync_copy(gbuf, acc.at[cv.at[k0]], add=True)

            @pl.when(g < SCHUNKS // 2 - 1)
            def _():
                gather(rv, k0 + 2, gbuf, gsem)
            gwait(rv, k0 + 1, gbuf2, gsem2)
            pltpu.sync_copy(gbuf2, acc.at[cv.at[k0 + 1]], add=True)
            return carry
        lax.fori_loop(0, SCHUNKS // 2, pair_body, 0)
    plsc.subcore_barrier()

    @pl.when(s < NUM_TILES - 1)
    def _():
        pltpu.sync_copy(acc.at[pl.ds(nbase, ROWS_PER_TILE)],
                        out_hbm.at[pl.ds(c * N_NODES + nbase, ROWS_PER_TILE)])

    @pl.when(s == NUM_TILES - 1)
    def _():
        pltpu.sync_copy(acc.at[pl.ds(nbase, LAST_VALID)],
                        out_hbm.at[pl.ds(c * N_NODES + nbase, LAST_VALID)])


_mesh = plsc.VectorSubcoreMesh(core_axis_name="c", subcore_axis_name="s")

_sc_call = functools.partial(
    pl.kernel,
    out_type=jax.ShapeDtypeStruct((BATCH * N_NODES, D_FEAT), jnp.float32),
    mesh=_mesh,
    scratch_types=[
        pltpu.VMEM((SCHUNKS, CHUNK), jnp.int32),
        pltpu.VMEM((SCHUNKS, CHUNK), jnp.int32),
        pltpu.VMEM((SCHUNKS, CHUNK), jnp.int32),
        pltpu.VMEM((SCHUNKS, CHUNK), jnp.int32),
        pltpu.VMEM((CHUNK, D_FEAT), jnp.float32),
        pltpu.VMEM((CHUNK, D_FEAT), jnp.float32),
        pltpu.VMEM_SHARED((ACC_ROWS, D_FEAT), jnp.float32),
        pltpu.SemaphoreType.DMA,
        pltpu.SemaphoreType.DMA,
        pltpu.SemaphoreType.DMA,
        pltpu.SemaphoreType.DMA,
    ],
)(_sc_body)


def kernel(t, edge_index):
    b, n, d = t.shape
    t2 = t.reshape(b * n, d)
    row = edge_index[0].reshape(NUM_TILES, E_RAW_PER_TILE)
    col = edge_index[1].reshape(NUM_TILES, E_RAW_PER_TILE)
    row_pad = jnp.zeros((NUM_TILES, PAD_PER_TILE), jnp.int32)
    col_pad = jnp.broadcast_to(
        N_NODES + (jnp.arange(PAD_PER_TILE, dtype=jnp.int32) % TRASH),
        (NUM_TILES, PAD_PER_TILE))
    rowp = jnp.concatenate([row, row_pad], axis=1)
    colp = jnp.concatenate([col, col_pad], axis=1)
    row_b = rowp[None] + (jnp.arange(b, dtype=jnp.int32) * n).reshape(b, 1, 1)
    row5 = row_b.reshape(b, NUM_TILES, N_STAGES, SCHUNKS, CHUNK)
    col4 = colp.reshape(NUM_TILES, N_STAGES, SCHUNKS, CHUNK)
    out2 = _sc_call(t2, row5, col4)
    return out2.reshape(b, n, d)


# speedup vs baseline: 1.0174x; 1.0174x over previous
import functools

import jax
import jax.numpy as jnp
from jax import lax
from jax.experimental import pallas as pl
from jax.experimental.pallas import tpu as pltpu
from jax.experimental.pallas import tpu_sc as plsc

N_NODES = 10000
N_EDGES = 320000
D_FEAT = 128
BATCH = 2
LANES = 16

NUM_TILES = 16
CHUNK = 128
N_STAGES = 8
SCHUNKS = 20
E_PER_TILE = N_STAGES * SCHUNKS * CHUNK
E_RAW_PER_TILE = N_EDGES // NUM_TILES
PAD_PER_TILE = E_PER_TILE - E_RAW_PER_TILE

TRASH = 8
ACC_ROWS = N_NODES + TRASH
ROWS_PER_TILE = 632
LAST_ROWS = ACC_ROWS - 15 * ROWS_PER_TILE
LAST_VALID = N_NODES - 15 * ROWS_PER_TILE


def _sc_body(t_hbm, row_hbm, col_hbm, out_hbm,
             row_v, col_v, row_v2, col_v2, gbuf, gbuf2, acc,
             gsem, gsem2, isem, isem2):
    c = lax.axis_index("c")
    s = lax.axis_index("s")
    nbase = s * ROWS_PER_TILE

    def zrow(i, carry):
        for j in range(D_FEAT // LANES):
            gbuf[i, pl.ds(j * LANES, LANES)] = jnp.zeros((LANES,), jnp.float32)
        return carry
    lax.fori_loop(0, CHUNK, zrow, 0)

    @pl.when(s < NUM_TILES - 1)
    def _():
        for i in range(4):
            pltpu.sync_copy(gbuf, acc.at[pl.ds(nbase + i * CHUNK, CHUNK)])
        pltpu.sync_copy(gbuf.at[pl.ds(0, ROWS_PER_TILE - 4 * CHUNK)],
                        acc.at[pl.ds(nbase + 4 * CHUNK, ROWS_PER_TILE - 4 * CHUNK)])

    @pl.when(s == NUM_TILES - 1)
    def _():
        for i in range(4):
            pltpu.sync_copy(gbuf, acc.at[pl.ds(nbase + i * CHUNK, CHUNK)])
        pltpu.sync_copy(gbuf.at[pl.ds(0, LAST_ROWS - 4 * CHUNK)],
                        acc.at[pl.ds(nbase + 4 * CHUNK, LAST_ROWS - 4 * CHUNK)])

    plsc.subcore_barrier()

    def gather(rv, k, buf, sem):
        pltpu.async_copy(t_hbm.at[rv.at[k]], buf, sem)

    def gwait(rv, k, buf, sem):
        pltpu.make_async_copy(t_hbm.at[rv.at[k]], buf, sem).wait()

    def iload(q, rv, cv, sem):
        pltpu.async_copy(row_hbm.at[c, s, q], rv, sem)
        pltpu.async_copy(col_hbm.at[s, q], cv, sem)

    def iwait(q, rv, cv, sem):
        pltpu.make_async_copy(row_hbm.at[c, s, q], rv, sem).wait()
        pltpu.make_async_copy(col_hbm.at[s, q], cv, sem).wait()

    bufs = [(row_v, col_v, isem), (row_v2, col_v2, isem2)]
    iload(0, *bufs[0])
    for q in range(N_STAGES):
        rv, cv, isem_q = bufs[q % 2]
        if q == 0:
            iwait(q, rv, cv, isem_q)
            iload(1, *bufs[1])
            gather(rv, 0, gbuf, gsem)

        def pair_body(g, carry):
            k0 = 2 * g
            gather(rv, k0 + 1, gbuf2, gsem2)
            gwait(rv, k0, gbuf, gsem)
            pltpu.sync_copy(gbuf, acc.at[cv.at[k0]], add=True)
            gather(rv, k0 + 2, gbuf, gsem)
            gwait(rv, k0 + 1, gbuf2, gsem2)
            pltpu.sync_copy(gbuf2, acc.at[cv.at[k0 + 1]], add=True)
            return carry
        lax.fori_loop(0, SCHUNKS // 2 - 1, pair_body, 0)

        kl = SCHUNKS - 2
        gather(rv, kl + 1, gbuf2, gsem2)
        gwait(rv, kl, gbuf, gsem)
        pltpu.sync_copy(gbuf, acc.at[cv.at[kl]], add=True)
        if q + 1 < N_STAGES:
            rvn, cvn, isem_n = bufs[(q + 1) % 2]
            iwait(q + 1, rvn, cvn, isem_n)
            gather(rvn, 0, gbuf, gsem)
        gwait(rv, kl + 1, gbuf2, gsem2)
        pltpu.sync_copy(gbuf2, acc.at[cv.at[kl + 1]], add=True)
        if q + 2 < N_STAGES:
            iload(q + 2, rv, cv, isem_q)
    plsc.subcore_barrier()

    @pl.when(s < NUM_TILES - 1)
    def _():
        pltpu.sync_copy(acc.at[pl.ds(nbase, ROWS_PER_TILE)],
                        out_hbm.at[pl.ds(c * N_NODES + nbase, ROWS_PER_TILE)])

    @pl.when(s == NUM_TILES - 1)
    def _():
        pltpu.sync_copy(acc.at[pl.ds(nbase, LAST_VALID)],
                        out_hbm.at[pl.ds(c * N_NODES + nbase, LAST_VALID)])


_mesh = plsc.VectorSubcoreMesh(core_axis_name="c", subcore_axis_name="s")

_sc_call = functools.partial(
    pl.kernel,
    out_type=jax.ShapeDtypeStruct((BATCH * N_NODES, D_FEAT), jnp.float32),
    mesh=_mesh,
    scratch_types=[
        pltpu.VMEM((SCHUNKS, CHUNK), jnp.int32),
        pltpu.VMEM((SCHUNKS, CHUNK), jnp.int32),
        pltpu.VMEM((SCHUNKS, CHUNK), jnp.int32),
        pltpu.VMEM((SCHUNKS, CHUNK), jnp.int32),
        pltpu.VMEM((CHUNK, D_FEAT), jnp.float32),
        pltpu.VMEM((CHUNK, D_FEAT), jnp.float32),
        pltpu.VMEM_SHARED((ACC_ROWS, D_FEAT), jnp.float32),
        pltpu.SemaphoreType.DMA,
        pltpu.SemaphoreType.DMA,
        pltpu.SemaphoreType.DMA,
        pltpu.SemaphoreType.DMA,
    ],
)(_sc_body)


def kernel(t, edge_index):
    b, n, d = t.shape
    t2 = t.reshape(b * n, d)
    row = edge_index[0].reshape(NUM_TILES, E_RAW_PER_TILE)
    col = edge_index[1].reshape(NUM_TILES, E_RAW_PER_TILE)
    row_pad = jnp.zeros((NUM_TILES, PAD_PER_TILE), jnp.int32)
    col_pad = jnp.broadcast_to(
        N_NODES + (jnp.arange(PAD_PER_TILE, dtype=jnp.int32) % TRASH),
        (NUM_TILES, PAD_PER_TILE))
    rowp = jnp.concatenate([row, row_pad], axis=1)
    colp = jnp.concatenate([col, col_pad], axis=1)
    row_b = rowp[None] + (jnp.arange(b, dtype=jnp.int32) * n).reshape(b, 1, 1)
    row5 = row_b.reshape(b, NUM_TILES, N_STAGES, SCHUNKS, CHUNK)
    col4 = colp.reshape(NUM_TILES, N_STAGES, SCHUNKS, CHUNK)
    out2 = _sc_call(t2, row5, col4)
    return out2.reshape(b, n, d)


# speedup vs baseline: 1.0223x; 1.0048x over previous
import functools

import jax
import jax.numpy as jnp
from jax import lax
from jax.experimental import pallas as pl
from jax.experimental.pallas import tpu as pltpu
from jax.experimental.pallas import tpu_sc as plsc

N_NODES = 10000
N_EDGES = 320000
D_FEAT = 128
BATCH = 2
LANES = 16

NUM_TILES = 16
CHUNK = 128
N_STAGES = 8
SCHUNKS = 20
E_PER_TILE = N_STAGES * SCHUNKS * CHUNK
E_RAW_PER_TILE = N_EDGES // NUM_TILES
PAD_PER_TILE = E_PER_TILE - E_RAW_PER_TILE

TRASH = 8
ACC_ROWS = N_NODES + TRASH
ROWS_PER_TILE = 632
LAST_ROWS = ACC_ROWS - 15 * ROWS_PER_TILE
LAST_VALID = N_NODES - 15 * ROWS_PER_TILE


def _sc_body(t_hbm, row_hbm, col_hbm, out_hbm,
             row_v, col_v, row_v2, col_v2, gbuf, gbuf2, acc,
             gsem, gsem2, isem, isem2):
    c = lax.axis_index("c")
    s = lax.axis_index("s")
    nbase = s * ROWS_PER_TILE

    pltpu.async_copy(row_hbm.at[c, s, 0], row_v, isem)
    pltpu.async_copy(col_hbm.at[s, 0], col_v, isem)
    pltpu.async_copy(row_hbm.at[c, s, 1], row_v2, isem2)
    pltpu.async_copy(col_hbm.at[s, 1], col_v2, isem2)

    def zrow(i, carry):
        for j in range(D_FEAT // LANES):
            gbuf[i, pl.ds(j * LANES, LANES)] = jnp.zeros((LANES,), jnp.float32)
        return carry
    lax.fori_loop(0, CHUNK, zrow, 0)

    for i in range(4):
        pltpu.async_copy(gbuf, acc.at[pl.ds(nbase + i * CHUNK, CHUNK)], gsem)
    for i in range(4):
        pltpu.make_async_copy(gbuf, acc.at[pl.ds(nbase + i * CHUNK, CHUNK)],
                              gsem).wait()

    @pl.when(s < NUM_TILES - 1)
    def _():
        pltpu.sync_copy(gbuf.at[pl.ds(0, ROWS_PER_TILE - 4 * CHUNK)],
                        acc.at[pl.ds(nbase + 4 * CHUNK, ROWS_PER_TILE - 4 * CHUNK)])

    @pl.when(s == NUM_TILES - 1)
    def _():
        pltpu.sync_copy(gbuf.at[pl.ds(0, LAST_ROWS - 4 * CHUNK)],
                        acc.at[pl.ds(nbase + 4 * CHUNK, LAST_ROWS - 4 * CHUNK)])

    plsc.subcore_barrier()

    def gather(rv, k, buf, sem):
        pltpu.async_copy(t_hbm.at[rv.at[k]], buf, sem)

    def gwait(rv, k, buf, sem):
        pltpu.make_async_copy(t_hbm.at[rv.at[k]], buf, sem).wait()

    def iload(q, rv, cv, sem):
        pltpu.async_copy(row_hbm.at[c, s, q], rv, sem)
        pltpu.async_copy(col_hbm.at[s, q], cv, sem)

    def iwait(q, rv, cv, sem):
        pltpu.make_async_copy(row_hbm.at[c, s, q], rv, sem).wait()
        pltpu.make_async_copy(col_hbm.at[s, q], cv, sem).wait()

    bufs = [(row_v, col_v, isem), (row_v2, col_v2, isem2)]
    for q in range(N_STAGES):
        rv, cv, isem_q = bufs[q % 2]
        if q == 0:
            iwait(q, rv, cv, isem_q)
            gather(rv, 0, gbuf, gsem)

        def pair_body(g, carry):
            k0 = 2 * g
            gather(rv, k0 + 1, gbuf2, gsem2)
            gwait(rv, k0, gbuf, gsem)
            pltpu.sync_copy(gbuf, acc.at[cv.at[k0]], add=True)
            gather(rv, k0 + 2, gbuf, gsem)
            gwait(rv, k0 + 1, gbuf2, gsem2)
            pltpu.sync_copy(gbuf2, acc.at[cv.at[k0 + 1]], add=True)
            return carry
        lax.fori_loop(0, SCHUNKS // 2 - 1, pair_body, 0)

        kl = SCHUNKS - 2
        gather(rv, kl + 1, gbuf2, gsem2)
        gwait(rv, kl, gbuf, gsem)
        pltpu.sync_copy(gbuf, acc.at[cv.at[kl]], add=True)
        if q + 1 < N_STAGES:
            rvn, cvn, isem_n = bufs[(q + 1) % 2]
            iwait(q + 1, rvn, cvn, isem_n)
            gather(rvn, 0, gbuf, gsem)
        gwait(rv, kl + 1, gbuf2, gsem2)
        pltpu.sync_copy(gbuf2, acc.at[cv.at[kl + 1]], add=True)
        if q + 2 < N_STAGES:
            iload(q + 2, rv, cv, isem_q)
    plsc.subcore_barrier()

    @pl.when(s < NUM_TILES - 1)
    def _():
        pltpu.sync_copy(acc.at[pl.ds(nbase, ROWS_PER_TILE)],
                        out_hbm.at[pl.ds(c * N_NODES + nbase, ROWS_PER_TILE)])

    @pl.when(s == NUM_TILES - 1)
    def _():
        pltpu.sync_copy(acc.at[pl.ds(nbase, LAST_VALID)],
                        out_hbm.at[pl.ds(c * N_NODES + nbase, LAST_VALID)])


_mesh = plsc.VectorSubcoreMesh(core_axis_name="c", subcore_axis_name="s")

_sc_call = functools.partial(
    pl.kernel,
    out_type=jax.ShapeDtypeStruct((BATCH * N_NODES, D_FEAT), jnp.float32),
    mesh=_mesh,
    scratch_types=[
        pltpu.VMEM((SCHUNKS, CHUNK), jnp.int32),
        pltpu.VMEM((SCHUNKS, CHUNK), jnp.int32),
        pltpu.VMEM((SCHUNKS, CHUNK), jnp.int32),
        pltpu.VMEM((SCHUNKS, CHUNK), jnp.int32),
        pltpu.VMEM((CHUNK, D_FEAT), jnp.float32),
        pltpu.VMEM((CHUNK, D_FEAT), jnp.float32),
        pltpu.VMEM_SHARED((ACC_ROWS, D_FEAT), jnp.float32),
        pltpu.SemaphoreType.DMA,
        pltpu.SemaphoreType.DMA,
        pltpu.SemaphoreType.DMA,
        pltpu.SemaphoreType.DMA,
    ],
)(_sc_body)


def kernel(t, edge_index):
    b, n, d = t.shape
    t2 = t.reshape(b * n, d)
    row = edge_index[0].reshape(NUM_TILES, E_RAW_PER_TILE)
    col = edge_index[1].reshape(NUM_TILES, E_RAW_PER_TILE)
    row_pad = jnp.zeros((NUM_TILES, PAD_PER_TILE), jnp.int32)
    col_pad = jnp.broadcast_to(
        N_NODES + (jnp.arange(PAD_PER_TILE, dtype=jnp.int32) % TRASH),
        (NUM_TILES, PAD_PER_TILE))
    rowp = jnp.concatenate([row, row_pad], axis=1)
    colp = jnp.concatenate([col, col_pad], axis=1)
    row_b = rowp[None] + (jnp.arange(b, dtype=jnp.int32) * n).reshape(b, 1, 1)
    row5 = row_b.reshape(b, NUM_TILES, N_STAGES, SCHUNKS, CHUNK)
    col4 = colp.reshape(NUM_TILES, N_STAGES, SCHUNKS, CHUNK)
    out2 = _sc_call(t2, row5, col4)
    return out2.reshape(b, n, d)
